# Initial kernel scaffold; baseline (speedup 1.0000x reference)
import jax
import jax.numpy as jnp
from jax.experimental import pallas as pl


def kernel(x, edge_index, W1, b1, W2, b2):
    raise NotImplementedError("write your pallas kernel here")


# speedup vs baseline: 7.1848x; 7.1848x over previous
import functools

import jax
import jax.numpy as jnp
from jax import lax
from jax.experimental import pallas as pl
from jax.experimental.pallas import tpu as pltpu
from jax.experimental.pallas import tpu_sc as plsc

N = 10000
D = 256
HD = 128
E = 160000
NPAD = 10240
TILES = 16
CORES = 2
CHUNK = 128
NCH = 80
E_PAD = TILES * NCH * CHUNK
STRIPE = NPAD // TILES
PASSES = 2

_SC_MESH = plsc.VectorSubcoreMesh(
    core_axis_name="c", subcore_axis_name="s", num_cores=CORES,
    num_subcores=TILES)


def _deg_body(dst_hbm, z128_hbm, ones_hbm, deg0_hbm, deg1_hbm,
              deg_sh, dstv, ones_v):
    c = lax.axis_index("c")
    s = lax.axis_index("s")
    row0 = s * STRIPE
    sl = pl.ds(row0, STRIPE)
    pltpu.sync_copy(z128_hbm, deg_sh.at[sl])
    pltpu.sync_copy(ones_hbm, ones_v)
    pltpu.sync_copy(dst_hbm.at[s], dstv)
    plsc.subcore_barrier()

    def count_half(lo):
        def body(j, carry):
            pltpu.sync_copy(ones_v, deg_sh.at[dstv.at[lo + j]], add=True)
            return carry

        lax.fori_loop(0, NCH // 2, body, 0)

    @pl.when(c == 0)
    def _():
        count_half(0)

    @pl.when(c == 1)
    def _():
        count_half(NCH // 2)

    plsc.subcore_barrier()

    @pl.when(c == 0)
    def _():
        pltpu.sync_copy(deg_sh.at[sl], deg0_hbm.at[sl])

    @pl.when(c == 1)
    def _():
        pltpu.sync_copy(deg_sh.at[sl], deg1_hbm.at[sl])


_deg_call = pl.kernel(
    _deg_body,
    out_type=(jax.ShapeDtypeStruct((NPAD, HD), jnp.float32),
              jax.ShapeDtypeStruct((NPAD, HD), jnp.float32)),
    mesh=_SC_MESH,
    scratch_types=[
        pltpu.VMEM_SHARED((NPAD, HD), jnp.float32),
        pltpu.VMEM((NCH, CHUNK), jnp.int32),
        pltpu.VMEM((CHUNK, HD), jnp.float32),
    ],
)


def _agg_body(g0_hbm, g1_hbm, src_hbm, dst_hbm, z128_hbm, s0_hbm, s1_hbm,
              acc_sh, srcv, dstv, msg_a, msg_b, sem_a, sem_b):
    c = lax.axis_index("c")
    s = lax.axis_index("s")
    row0 = s * STRIPE
    sl = pl.ds(row0, STRIPE)
    pltpu.sync_copy(z128_hbm, acc_sh.at[sl])
    plsc.subcore_barrier()

    def run(ghbm):
        for p in range(PASSES):
            npc = NCH // PASSES
            pltpu.sync_copy(src_hbm.at[s, pl.ds(p * npc, npc)], srcv)
            pltpu.sync_copy(dst_hbm.at[s, pl.ds(p * npc, npc)], dstv)
            pltpu.async_copy(ghbm.at[srcv.at[0]], msg_a, sem_a)

            def body(j, carry):
                j2 = 2 * j
                pltpu.make_async_copy(
                    ghbm.at[srcv.at[j2]], msg_a, sem_a).wait()
                cpb = pltpu.async_copy(
                    ghbm.at[srcv.at[j2 + 1]], msg_b, sem_b)
                pltpu.sync_copy(msg_a, acc_sh.at[dstv.at[j2]], add=True)
                cpb.wait()

                @pl.when(j2 + 2 < npc)
                def _():
                    pltpu.async_copy(ghbm.at[srcv.at[j2 + 2]], msg_a, sem_a)

                pltpu.sync_copy(msg_b, acc_sh.at[dstv.at[j2 + 1]], add=True)
                return carry

            lax.fori_loop(0, npc // 2, body, 0)

    @pl.when(c == 0)
    def _():
        run(g0_hbm)

    @pl.when(c == 1)
    def _():
        run(g1_hbm)

    plsc.subcore_barrier()

    @pl.when(c == 0)
    def _():
        pltpu.sync_copy(acc_sh.at[sl], s0_hbm.at[sl])

    @pl.when(c == 1)
    def _():
        pltpu.sync_copy(acc_sh.at[sl], s1_hbm.at[sl])


_agg_call = pl.kernel(
    _agg_body,
    out_type=(jax.ShapeDtypeStruct((NPAD, HD), jnp.float32),
              jax.ShapeDtypeStruct((NPAD, HD), jnp.float32)),
    mesh=_SC_MESH,
    scratch_types=[
        pltpu.VMEM_SHARED((NPAD, HD), jnp.float32),
        pltpu.VMEM((NCH // PASSES, CHUNK), jnp.int32),
        pltpu.VMEM((NCH // PASSES, CHUNK), jnp.int32),
        pltpu.VMEM((CHUNK, HD), jnp.float32),
        pltpu.VMEM((CHUNK, HD), jnp.float32),
        pltpu.SemaphoreType.DMA,
        pltpu.SemaphoreType.DMA,
    ],
)


R = 400
_GRID = (N // R,)


def _dis(d0_ref, d1_ref):
    return lax.rsqrt(d0_ref[...] + d1_ref[...] + 1.0)


def _mm1_body(x_ref, d0_ref, d1_ref, w_ref, o0_ref, o1_ref):
    g = jnp.dot(_dis(d0_ref, d1_ref) * x_ref[...], w_ref[...],
                preferred_element_type=jnp.float32)
    o0_ref[...] = g[:, :HD]
    o1_ref[...] = g[:, HD:]


def _mm2_body(s0_ref, s1_ref, g0_ref, g1_ref, d0_ref, d1_ref, b_ref, w_ref,
              o0_ref, o1_ref):
    dis = _dis(d0_ref, d1_ref)
    t = jnp.concatenate([s0_ref[...] + g0_ref[...],
                         s1_ref[...] + g1_ref[...]], axis=1)
    u = jnp.maximum(dis * t + b_ref[...], 0.0)
    g = jnp.dot(dis * u, w_ref[...], preferred_element_type=jnp.float32)
    o0_ref[...] = g[:, :HD]
    o1_ref[...] = g[:, HD:]


def _fin_body(s0_ref, s1_ref, g0_ref, g1_ref, d0_ref, d1_ref, b_ref,
              o0_ref, o1_ref):
    dis = _dis(d0_ref, d1_ref)
    o0_ref[...] = jnp.maximum(dis * (s0_ref[...] + g0_ref[...])
                              + b_ref[:, :HD], 0.0)
    o1_ref[...] = jnp.maximum(dis * (s1_ref[...] + g1_ref[...])
                              + b_ref[:, HD:], 0.0)


def _row_spec(cols):
    return pl.BlockSpec((R, cols), lambda i: (i, 0))


def _full_spec(rows, cols):
    return pl.BlockSpec((rows, cols), lambda i: (0, 0))


_mm1_call = pl.pallas_call(
    _mm1_body,
    grid=_GRID,
    in_specs=[_row_spec(D), _row_spec(1), _row_spec(1), _full_spec(D, D)],
    out_specs=[_row_spec(HD), _row_spec(HD)],
    out_shape=(jax.ShapeDtypeStruct((N, HD), jnp.float32),
               jax.ShapeDtypeStruct((N, HD), jnp.float32)),
)

_mm2_call = pl.pallas_call(
    _mm2_body,
    grid=_GRID,
    in_specs=[_row_spec(HD), _row_spec(HD), _row_spec(HD), _row_spec(HD),
              _row_spec(1), _row_spec(1), _full_spec(1, D), _full_spec(D, D)],
    out_specs=[_row_spec(HD), _row_spec(HD)],
    out_shape=(jax.ShapeDtypeStruct((N, HD), jnp.float32),
               jax.ShapeDtypeStruct((N, HD), jnp.float32)),
)

_fin_call = pl.pallas_call(
    _fin_body,
    grid=_GRID,
    in_specs=[_row_spec(HD), _row_spec(HD), _row_spec(HD), _row_spec(HD),
              _row_spec(1), _row_spec(1), _full_spec(1, D)],
    out_specs=[_row_spec(HD), _row_spec(HD)],
    out_shape=(jax.ShapeDtypeStruct((N, HD), jnp.float32),
               jax.ShapeDtypeStruct((N, HD), jnp.float32)),
)


def kernel(x, edge_index, W1, b1, W2, b2):
    src = edge_index[0].astype(jnp.int32)
    dst = edge_index[1].astype(jnp.int32)
    src3 = jnp.concatenate(
        [src, jnp.zeros((E_PAD - E,), jnp.int32)]).reshape(TILES, NCH, CHUNK)
    dst3 = jnp.concatenate(
        [dst, jnp.full((E_PAD - E,), N, jnp.int32)]).reshape(TILES, NCH, CHUNK)

    z128 = jnp.zeros((STRIPE, HD), jnp.float32)
    ones128 = jnp.ones((CHUNK, HD), jnp.float32)

    deg0, deg1 = _deg_call(dst3, z128, ones128)
    d0 = deg0[:N, 0:1]
    d1 = deg1[:N, 0:1]

    g0, g1 = _mm1_call(x, d0, d1, W1)
    s0, s1 = _agg_call(g0, g1, src3, dst3, z128)
    h0, h1 = _mm2_call(s0[:N], s1[:N], g0, g1, d0, d1, b1.reshape(1, D), W2)
    t0, t1 = _agg_call(h0, h1, src3, dst3, z128)
    o0, o1 = _fin_call(t0[:N], t1[:N], h0, h1, d0, d1, b2.reshape(1, D))
    return jnp.concatenate([o0, o1], axis=1)


# speedup vs baseline: 7.3633x; 1.0248x over previous
import functools

import jax
import jax.numpy as jnp
from jax import lax
from jax.experimental import pallas as pl
from jax.experimental.pallas import tpu as pltpu
from jax.experimental.pallas import tpu_sc as plsc

N = 10000
D = 256
HD = 128
E = 160000
NPAD = 10240
TILES = 16
CORES = 2
CHUNK = 64
NCH = 160
E_PAD = TILES * NCH * CHUNK
STRIPE = NPAD // TILES
PASSES = 4
NB = 4

_SC_MESH = plsc.VectorSubcoreMesh(
    core_axis_name="c", subcore_axis_name="s", num_cores=CORES,
    num_subcores=TILES)


def _deg_body(dst_hbm, z128_hbm, ones_hbm, deg0_hbm, deg1_hbm,
              deg_sh, dstv, ones_v, semd):
    c = lax.axis_index("c")
    s = lax.axis_index("s")
    row0 = s * STRIPE
    sl = pl.ds(row0, STRIPE)
    pltpu.sync_copy(z128_hbm, deg_sh.at[sl])
    pltpu.sync_copy(ones_hbm, ones_v)
    pltpu.sync_copy(dst_hbm.at[s], dstv)
    plsc.subcore_barrier()

    def count_half(lo):
        def body(gi, carry):
            for b in range(8):
                pltpu.async_copy(ones_v, deg_sh.at[dstv.at[lo + gi * 8 + b]],
                                 semd, add=True)
            for b in range(8):
                pltpu.make_async_copy(
                    ones_v, deg_sh.at[dstv.at[lo + gi * 8 + b]], semd).wait()
            return carry

        lax.fori_loop(0, NCH // 16, body, 0)

    @pl.when(c == 0)
    def _():
        count_half(0)

    @pl.when(c == 1)
    def _():
        count_half(NCH // 2)

    plsc.subcore_barrier()

    @pl.when(c == 0)
    def _():
        pltpu.sync_copy(deg_sh.at[sl], deg0_hbm.at[sl])

    @pl.when(c == 1)
    def _():
        pltpu.sync_copy(deg_sh.at[sl], deg1_hbm.at[sl])


_deg_call = pl.kernel(
    _deg_body,
    out_type=(jax.ShapeDtypeStruct((NPAD, HD), jnp.float32),
              jax.ShapeDtypeStruct((NPAD, HD), jnp.float32)),
    mesh=_SC_MESH,
    scratch_types=[
        pltpu.VMEM_SHARED((NPAD, HD), jnp.float32),
        pltpu.VMEM((NCH, CHUNK), jnp.int32),
        pltpu.VMEM((CHUNK, HD), jnp.float32),
        pltpu.SemaphoreType.DMA,
    ],
)


def _agg_body(g0_hbm, g1_hbm, src_hbm, dst_hbm, z128_hbm, s0_hbm, s1_hbm,
              acc_sh, srcv, dstv, m0, m1, m2, m3, sg0, sg1, sg2, sg3,
              ss0, ss1, ss2, ss3):
    msg = [m0, m1, m2, m3]
    semg = [sg0, sg1, sg2, sg3]
    sems = [ss0, ss1, ss2, ss3]
    c = lax.axis_index("c")
    s = lax.axis_index("s")
    row0 = s * STRIPE
    sl = pl.ds(row0, STRIPE)
    pltpu.sync_copy(z128_hbm, acc_sh.at[sl])
    plsc.subcore_barrier()

    def run(ghbm):
        npc = NCH // PASSES
        for p in range(PASSES):
            pltpu.sync_copy(src_hbm.at[s, pl.ds(p * npc, npc)], srcv)
            pltpu.sync_copy(dst_hbm.at[s, pl.ds(p * npc, npc)], dstv)
            for b in range(NB):
                pltpu.async_copy(ghbm.at[srcv.at[b]], msg[b], semg[b])

            def group(gi, carry):
                for b in range(NB):
                    j = gi * NB + b
                    pltpu.make_async_copy(
                        ghbm.at[srcv.at[j]], msg[b], semg[b]).wait()
                    pltpu.async_copy(msg[b], acc_sh.at[dstv.at[j]], sems[b],
                                     add=True)
                for b in range(NB):
                    j = gi * NB + b
                    pltpu.make_async_copy(
                        msg[b], acc_sh.at[dstv.at[j]], sems[b]).wait()

                    @pl.when(j + NB < npc)
                    def _(b=b, j=j):
                        pltpu.async_copy(
                            ghbm.at[srcv.at[j + NB]], msg[b], semg[b])
                return carry

            lax.fori_loop(0, npc // NB, group, 0)

    @pl.when(c == 0)
    def _():
        run(g0_hbm)

    @pl.when(c == 1)
    def _():
        run(g1_hbm)

    plsc.subcore_barrier()

    @pl.when(c == 0)
    def _():
        pltpu.sync_copy(acc_sh.at[sl], s0_hbm.at[sl])

    @pl.when(c == 1)
    def _():
        pltpu.sync_copy(acc_sh.at[sl], s1_hbm.at[sl])


_agg_call = pl.kernel(
    _agg_body,
    out_type=(jax.ShapeDtypeStruct((NPAD, HD), jnp.float32),
              jax.ShapeDtypeStruct((NPAD, HD), jnp.float32)),
    mesh=_SC_MESH,
    scratch_types=[
        pltpu.VMEM_SHARED((NPAD, HD), jnp.float32),
        pltpu.VMEM((NCH // PASSES, CHUNK), jnp.int32),
        pltpu.VMEM((NCH // PASSES, CHUNK), jnp.int32),
        pltpu.VMEM((CHUNK, HD), jnp.float32),
        pltpu.VMEM((CHUNK, HD), jnp.float32),
        pltpu.VMEM((CHUNK, HD), jnp.float32),
        pltpu.VMEM((CHUNK, HD), jnp.float32),
        pltpu.SemaphoreType.DMA,
        pltpu.SemaphoreType.DMA,
        pltpu.SemaphoreType.DMA,
        pltpu.SemaphoreType.DMA,
        pltpu.SemaphoreType.DMA,
        pltpu.SemaphoreType.DMA,
        pltpu.SemaphoreType.DMA,
        pltpu.SemaphoreType.DMA,
    ],
)


R = 400
_GRID = (N // R,)


def _dis(d0_ref, d1_ref):
    return lax.rsqrt(d0_ref[...] + d1_ref[...] + 1.0)


def _mm1_body(x_ref, d0_ref, d1_ref, w_ref, o0_ref, o1_ref):
    g = jnp.dot(_dis(d0_ref, d1_ref) * x_ref[...], w_ref[...],
                preferred_element_type=jnp.float32)
    o0_ref[...] = g[:, :HD]
    o1_ref[...] = g[:, HD:]


def _mm2_body(s0_ref, s1_ref, g0_ref, g1_ref, d0_ref, d1_ref, b_ref, w_ref,
              o0_ref, o1_ref):
    dis = _dis(d0_ref, d1_ref)
    t = jnp.concatenate([s0_ref[...] + g0_ref[...],
                         s1_ref[...] + g1_ref[...]], axis=1)
    u = jnp.maximum(dis * t + b_ref[...], 0.0)
    g = jnp.dot(dis * u, w_ref[...], preferred_element_type=jnp.float32)
    o0_ref[...] = g[:, :HD]
    o1_ref[...] = g[:, HD:]


def _fin_body(s0_ref, s1_ref, g0_ref, g1_ref, d0_ref, d1_ref, b_ref,
              o0_ref, o1_ref):
    dis = _dis(d0_ref, d1_ref)
    o0_ref[...] = jnp.maximum(dis * (s0_ref[...] + g0_ref[...])
                              + b_ref[:, :HD], 0.0)
    o1_ref[...] = jnp.maximum(dis * (s1_ref[...] + g1_ref[...])
                              + b_ref[:, HD:], 0.0)


def _row_spec(cols):
    return pl.BlockSpec((R, cols), lambda i: (i, 0))


def _full_spec(rows, cols):
    return pl.BlockSpec((rows, cols), lambda i: (0, 0))


_mm1_call = pl.pallas_call(
    _mm1_body,
    grid=_GRID,
    in_specs=[_row_spec(D), _row_spec(1), _row_spec(1), _full_spec(D, D)],
    out_specs=[_row_spec(HD), _row_spec(HD)],
    out_shape=(jax.ShapeDtypeStruct((N, HD), jnp.float32),
               jax.ShapeDtypeStruct((N, HD), jnp.float32)),
)

_mm2_call = pl.pallas_call(
    _mm2_body,
    grid=_GRID,
    in_specs=[_row_spec(HD), _row_spec(HD), _row_spec(HD), _row_spec(HD),
              _row_spec(1), _row_spec(1), _full_spec(1, D), _full_spec(D, D)],
    out_specs=[_row_spec(HD), _row_spec(HD)],
    out_shape=(jax.ShapeDtypeStruct((N, HD), jnp.float32),
               jax.ShapeDtypeStruct((N, HD), jnp.float32)),
)

_fin_call = pl.pallas_call(
    _fin_body,
    grid=_GRID,
    in_specs=[_row_spec(HD), _row_spec(HD), _row_spec(HD), _row_spec(HD),
              _row_spec(1), _row_spec(1), _full_spec(1, D)],
    out_specs=[_row_spec(HD), _row_spec(HD)],
    out_shape=(jax.ShapeDtypeStruct((N, HD), jnp.float32),
               jax.ShapeDtypeStruct((N, HD), jnp.float32)),
)


def kernel(x, edge_index, W1, b1, W2, b2):
    src = edge_index[0].astype(jnp.int32)
    dst = edge_index[1].astype(jnp.int32)
    src3 = jnp.concatenate(
        [src, jnp.zeros((E_PAD - E,), jnp.int32)]).reshape(TILES, NCH, CHUNK)
    dst3 = jnp.concatenate(
        [dst, jnp.full((E_PAD - E,), N, jnp.int32)]).reshape(TILES, NCH, CHUNK)

    z128 = jnp.zeros((STRIPE, HD), jnp.float32)
    ones128 = jnp.ones((CHUNK, HD), jnp.float32)

    deg0, deg1 = _deg_call(dst3, z128, ones128)
    d0 = deg0[:N, 0:1]
    d1 = deg1[:N, 0:1]

    g0, g1 = _mm1_call(x, d0, d1, W1)
    s0, s1 = _agg_call(g0, g1, src3, dst3, z128)
    h0, h1 = _mm2_call(s0[:N], s1[:N], g0, g1, d0, d1, b1.reshape(1, D), W2)
    t0, t1 = _agg_call(h0, h1, src3, dst3, z128)
    o0, o1 = _fin_call(t0[:N], t1[:N], h0, h1, d0, d1, b2.reshape(1, D))
    return jnp.concatenate([o0, o1], axis=1)


# speedup vs baseline: 7.7788x; 1.0564x over previous
import jax
import jax.numpy as jnp
from jax import lax
from jax.experimental import pallas as pl
from jax.experimental.pallas import tpu as pltpu
from jax.experimental.pallas import tpu_sc as plsc

N = 10000
D = 256
HD = 128
E = 160000
NPAD = 10240
TILES = 16
CORES = 2
CHUNK = 64
NCH = 160
E_PAD = TILES * NCH * CHUNK
STRIPE = NPAD // TILES
NB = 4
HCH = NCH // 2
PASSES = 4
PC = NCH // PASSES

_SC_MESH = plsc.VectorSubcoreMesh(
    core_axis_name="c", subcore_axis_name="s", num_cores=CORES,
    num_subcores=TILES)


def _deg_body(dst_hbm, z128_hbm, ones_hbm, deg0_hbm, deg1_hbm,
              deg_sh, dstv, ones_v, semd):
    c = lax.axis_index("c")
    s = lax.axis_index("s")
    sl = pl.ds(s * STRIPE, STRIPE)
    pltpu.sync_copy(z128_hbm, deg_sh.at[sl])
    pltpu.sync_copy(ones_hbm, ones_v)
    pltpu.sync_copy(dst_hbm.at[s], dstv)
    plsc.subcore_barrier()

    def count_half(lo):
        def body(gi, carry):
            for b in range(8):
                pltpu.async_copy(ones_v, deg_sh.at[dstv.at[lo + gi * 8 + b]],
                                 semd, add=True)
            for b in range(8):
                pltpu.make_async_copy(
                    ones_v, deg_sh.at[dstv.at[lo + gi * 8 + b]], semd).wait()
            return carry

        lax.fori_loop(0, HCH // 8, body, 0)

    @pl.when(c == 0)
    def _():
        count_half(0)

    @pl.when(c == 1)
    def _():
        count_half(HCH)

    plsc.subcore_barrier()

    @pl.when(c == 0)
    def _():
        pltpu.sync_copy(deg_sh.at[sl], deg0_hbm.at[sl])

    @pl.when(c == 1)
    def _():
        pltpu.sync_copy(deg_sh.at[sl], deg1_hbm.at[sl])


_deg_call = pl.kernel(
    _deg_body,
    out_type=(jax.ShapeDtypeStruct((NPAD, HD), jnp.float32),
              jax.ShapeDtypeStruct((NPAD, HD), jnp.float32)),
    mesh=_SC_MESH,
    scratch_types=[
        pltpu.VMEM_SHARED((NPAD, HD), jnp.float32),
        pltpu.VMEM((NCH, CHUNK), jnp.int32),
        pltpu.VMEM((CHUNK, HD), jnp.float32),
        pltpu.SemaphoreType.DMA,
    ],
)


def _agg_body(g0_hbm, g1_hbm, src_hbm, dst_hbm, z128_hbm, s0_hbm, s1_hbm,
              acc_sh, srcv, dstv, m0, m1, m2, m3, sg0, sg1, sg2, sg3,
              ss0, ss1, ss2, ss3):
    msg = [m0, m1, m2, m3]
    semg = [sg0, sg1, sg2, sg3]
    sems = [ss0, ss1, ss2, ss3]
    c = lax.axis_index("c")
    s = lax.axis_index("s")
    sl = pl.ds(s * STRIPE, STRIPE)
    pltpu.sync_copy(z128_hbm, acc_sh.at[sl])
    plsc.subcore_barrier()

    def run(ghbm):
        for p in range(PASSES):
            pltpu.sync_copy(src_hbm.at[s, pl.ds(p * PC, PC)], srcv)
            pltpu.sync_copy(dst_hbm.at[s, pl.ds(p * PC, PC)], dstv)
            for b in range(NB):
                pltpu.async_copy(ghbm.at[srcv.at[b]], msg[b], semg[b])

            def group(gi, carry):
                for b in range(NB):
                    j = gi * NB + b
                    pltpu.make_async_copy(
                        ghbm.at[srcv.at[j]], msg[b], semg[b]).wait()
                    pltpu.async_copy(msg[b], acc_sh.at[dstv.at[j]], sems[b],
                                     add=True)
                for b in range(NB):
                    j = gi * NB + b
                    pltpu.make_async_copy(
                        msg[b], acc_sh.at[dstv.at[j]], sems[b]).wait()

                    @pl.when(j + NB < PC)
                    def _(b=b, j=j):
                        pltpu.async_copy(
                            ghbm.at[srcv.at[j + NB]], msg[b], semg[b])
                return carry

            lax.fori_loop(0, PC // NB, group, 0)

    @pl.when(c == 0)
    def _():
        run(g0_hbm)

    @pl.when(c == 1)
    def _():
        run(g1_hbm)

    plsc.subcore_barrier()

    @pl.when(c == 0)
    def _():
        pltpu.sync_copy(acc_sh.at[sl], s0_hbm.at[sl])

    @pl.when(c == 1)
    def _():
        pltpu.sync_copy(acc_sh.at[sl], s1_hbm.at[sl])


_agg_call = pl.kernel(
    _agg_body,
    out_type=(jax.ShapeDtypeStruct((NPAD, HD), jnp.float32),
              jax.ShapeDtypeStruct((NPAD, HD), jnp.float32)),
    mesh=_SC_MESH,
    scratch_types=[
        pltpu.VMEM_SHARED((NPAD, HD), jnp.float32),
        pltpu.VMEM((PC, CHUNK), jnp.int32),
        pltpu.VMEM((PC, CHUNK), jnp.int32),
        pltpu.VMEM((CHUNK, HD), jnp.float32),
        pltpu.VMEM((CHUNK, HD), jnp.float32),
        pltpu.VMEM((CHUNK, HD), jnp.float32),
        pltpu.VMEM((CHUNK, HD), jnp.float32),
        pltpu.SemaphoreType.DMA,
        pltpu.SemaphoreType.DMA,
        pltpu.SemaphoreType.DMA,
        pltpu.SemaphoreType.DMA,
        pltpu.SemaphoreType.DMA,
        pltpu.SemaphoreType.DMA,
        pltpu.SemaphoreType.DMA,
        pltpu.SemaphoreType.DMA,
    ],
)


R = 400
_GRID = (N // R,)


def _dis(d0_ref, d1_ref):
    return lax.rsqrt(d0_ref[...] + d1_ref[...] + 1.0)


def _mmraw_body(x_ref, w_ref, o_ref):
    o_ref[...] = jnp.dot(x_ref[...], w_ref[...],
                         preferred_element_type=jnp.float32)


def _scale_body(h_ref, d0_ref, d1_ref, o0_ref, o1_ref):
    g = _dis(d0_ref, d1_ref) * h_ref[...]
    o0_ref[...] = g[:, :HD]
    o1_ref[...] = g[:, HD:]


def _mm2_body(s0_ref, s1_ref, g0_ref, g1_ref, d0_ref, d1_ref, b_ref, w_ref,
              o0_ref, o1_ref):
    dis = _dis(d0_ref, d1_ref)
    t = jnp.concatenate([s0_ref[...] + g0_ref[...],
                         s1_ref[...] + g1_ref[...]], axis=1)
    u = jnp.maximum(dis * t + b_ref[...], 0.0)
    g = jnp.dot(dis * u, w_ref[...], preferred_element_type=jnp.float32)
    o0_ref[...] = g[:, :HD]
    o1_ref[...] = g[:, HD:]


def _fin_body(s0_ref, s1_ref, g0_ref, g1_ref, d0_ref, d1_ref, b_ref,
              o0_ref, o1_ref):
    dis = _dis(d0_ref, d1_ref)
    o0_ref[...] = jnp.maximum(dis * (s0_ref[...] + g0_ref[...])
                              + b_ref[:, :HD], 0.0)
    o1_ref[...] = jnp.maximum(dis * (s1_ref[...] + g1_ref[...])
                              + b_ref[:, HD:], 0.0)


def _row_spec(cols):
    return pl.BlockSpec((R, cols), lambda i: (i, 0))


def _full_spec(rows, cols):
    return pl.BlockSpec((rows, cols), lambda i: (0, 0))


_OUT2 = (jax.ShapeDtypeStruct((N, HD), jnp.float32),
         jax.ShapeDtypeStruct((N, HD), jnp.float32))

_mmraw_call = pl.pallas_call(
    _mmraw_body,
    grid=_GRID,
    in_specs=[_row_spec(D), _full_spec(D, D)],
    out_specs=_row_spec(D),
    out_shape=jax.ShapeDtypeStruct((N, D), jnp.float32),
)

_scale_call = pl.pallas_call(
    _scale_body,
    grid=_GRID,
    in_specs=[_row_spec(D), _row_spec(1), _row_spec(1)],
    out_specs=[_row_spec(HD), _row_spec(HD)],
    out_shape=_OUT2,
)

_mm2_call = pl.pallas_call(
    _mm2_body,
    grid=_GRID,
    in_specs=[_row_spec(HD), _row_spec(HD), _row_spec(HD), _row_spec(HD),
              _row_spec(1), _row_spec(1), _full_spec(1, D), _full_spec(D, D)],
    out_specs=[_row_spec(HD), _row_spec(HD)],
    out_shape=_OUT2,
)

_fin_call = pl.pallas_call(
    _fin_body,
    grid=_GRID,
    in_specs=[_row_spec(HD), _row_spec(HD), _row_spec(HD), _row_spec(HD),
              _row_spec(1), _row_spec(1), _full_spec(1, D)],
    out_specs=[_row_spec(HD), _row_spec(HD)],
    out_shape=_OUT2,
)


def kernel(x, edge_index, W1, b1, W2, b2):
    src = edge_index[0].astype(jnp.int32)
    dst = edge_index[1].astype(jnp.int32)
    src3 = jnp.concatenate(
        [src, jnp.zeros((E_PAD - E,), jnp.int32)]).reshape(TILES, NCH, CHUNK)
    dst3 = jnp.concatenate(
        [dst, jnp.full((E_PAD - E,), N, jnp.int32)]).reshape(TILES, NCH, CHUNK)

    z128 = jnp.zeros((STRIPE, HD), jnp.float32)
    ones128 = jnp.ones((CHUNK, HD), jnp.float32)

    h1 = _mmraw_call(x, W1)
    deg0, deg1 = _deg_call(dst3, z128, ones128)
    d0 = deg0[:N, 0:1]
    d1 = deg1[:N, 0:1]

    g0, g1 = _scale_call(h1, d0, d1)
    s0, s1 = _agg_call(g0, g1, src3, dst3, z128)
    h0, h1b = _mm2_call(s0[:N], s1[:N], g0, g1, d0, d1,
                        b1.reshape(1, D), W2)
    t0, t1 = _agg_call(h0, h1b, src3, dst3, z128)
    o0, o1 = _fin_call(t0[:N], t1[:N], h0, h1b, d0, d1, b2.reshape(1, D))
    return jnp.concatenate([o0, o1], axis=1)
